# baseline (speedup 1.0000x reference)
import jax
import jax.numpy as jnp
from jax.experimental import pallas as pl
from jax.experimental.pallas import tpu as pltpu

B, S, D, E, K = 4, 8192, 1024, 64, 2
ROWS = 128
NBUF = 16
CPB = S // ROWS
NCHUNK = B * CPB


def _router_body(x_hbm, w_ref, b_ref, wout_ref, iout_ref, buf, acc_ref, sems):
    def start(t, k):
        pltpu.make_async_copy(
            x_hbm.at[pl.ds(t * ROWS, ROWS), :], buf.at[k], sems.at[k]
        ).start()

    def wait(k):
        pltpu.make_async_copy(
            x_hbm.at[pl.ds(0, ROWS), :], buf.at[k], sems.at[k]
        ).wait()

    for k in range(NBUF):
        start(k, k)

    acc_ref[...] = jnp.zeros_like(acc_ref)

    for b in range(B):
        def step(j, _, b=b):
            for k in range(NBUF):
                t = b * CPB + j * NBUF + k
                wait(k)
                acc_ref[b : b + 1, :] += jnp.sum(buf[k], axis=0, keepdims=True)

                @pl.when(t + NBUF < NCHUNK)
                def _():
                    start(t + NBUF, k)

            return 0

        jax.lax.fori_loop(0, CPB // NBUF, step, 0)

    pooled = acc_ref[...] * (1.0 / S)
    logits = jax.lax.dot_general(
        pooled, w_ref[...], (((1,), (1,)), ((), ())),
        preferred_element_type=jnp.float32,
    ) + b_ref[...]

    ids = jax.lax.broadcasted_iota(jnp.int32, (B, E), 1)
    neg_inf = jnp.float32(-jnp.inf)
    big = jnp.int32(E)

    v1 = jnp.max(logits, axis=1, keepdims=True)
    i1 = jnp.min(jnp.where(logits == v1, ids, big), axis=1, keepdims=True)
    masked = jnp.where(ids == i1, neg_inf, logits)
    v2 = jnp.max(masked, axis=1, keepdims=True)
    i2 = jnp.min(jnp.where(masked == v2, ids, big), axis=1, keepdims=True)

    e2 = jnp.exp(v2 - v1)
    w1 = 1.0 / (1.0 + e2)
    wout_ref[...] = jnp.concatenate([w1, e2 * w1], axis=1)
    iout_ref[...] = jnp.concatenate([i1, i2], axis=1)


@jax.jit
def kernel(x, gate_w, gate_b):
    weights, indices = pl.pallas_call(
        _router_body,
        in_specs=[
            pl.BlockSpec(memory_space=pl.ANY),
            pl.BlockSpec((E, D), lambda: (0, 0)),
            pl.BlockSpec((1, E), lambda: (0, 0)),
        ],
        out_specs=[
            pl.BlockSpec((B, K), lambda: (0, 0)),
            pl.BlockSpec((B, K), lambda: (0, 0)),
        ],
        out_shape=[
            jax.ShapeDtypeStruct((B, K), jnp.float32),
            jax.ShapeDtypeStruct((B, K), jnp.int32),
        ],
        scratch_shapes=[
            pltpu.VMEM((NBUF, ROWS, D), jnp.float32),
            pltpu.VMEM((B, D), jnp.float32),
            pltpu.SemaphoreType.DMA((NBUF,)),
        ],
    )(x.reshape(B * S, D), gate_w, gate_b.reshape(1, E))
    return weights, indices


# speedup vs baseline: 1.0300x; 1.0300x over previous
import jax
import jax.numpy as jnp
from jax.experimental import pallas as pl
from jax.experimental.pallas import tpu as pltpu

B, S, D, E, K = 4, 8192, 1024, 64, 2
ROWS = 512
NBUF = 4
CPB = S // ROWS
NCHUNK = B * CPB


def _router_body(x_hbm, w_ref, b_ref, wout_ref, iout_ref, buf, acc_ref, sems):
    def start(t, k):
        pltpu.make_async_copy(
            x_hbm.at[pl.ds(t * ROWS, ROWS), :], buf.at[k], sems.at[k]
        ).start()

    def wait(k):
        pltpu.make_async_copy(
            x_hbm.at[pl.ds(0, ROWS), :], buf.at[k], sems.at[k]
        ).wait()

    for k in range(NBUF):
        start(k, k)

    acc_ref[...] = jnp.zeros_like(acc_ref)

    for b in range(B):
        def step(j, _, b=b):
            for k in range(NBUF):
                t = b * CPB + j * NBUF + k
                wait(k)
                acc_ref[b : b + 1, :] += jnp.sum(buf[k], axis=0, keepdims=True)

                @pl.when(t + NBUF < NCHUNK)
                def _():
                    start(t + NBUF, k)

            return 0

        jax.lax.fori_loop(0, CPB // NBUF, step, 0)

    pooled = acc_ref[...] * (1.0 / S)
    logits = jax.lax.dot_general(
        pooled, w_ref[...], (((1,), (1,)), ((), ())),
        preferred_element_type=jnp.float32,
    ) + b_ref[...]

    ids = jax.lax.broadcasted_iota(jnp.int32, (B, E), 1)
    neg_inf = jnp.float32(-jnp.inf)
    big = jnp.int32(E)

    v1 = jnp.max(logits, axis=1, keepdims=True)
    i1 = jnp.min(jnp.where(logits == v1, ids, big), axis=1, keepdims=True)
    masked = jnp.where(ids == i1, neg_inf, logits)
    v2 = jnp.max(masked, axis=1, keepdims=True)
    i2 = jnp.min(jnp.where(masked == v2, ids, big), axis=1, keepdims=True)

    e2 = jnp.exp(v2 - v1)
    w1 = 1.0 / (1.0 + e2)
    wout_ref[...] = jnp.concatenate([w1, e2 * w1], axis=1)
    iout_ref[...] = jnp.concatenate([i1, i2], axis=1)


@jax.jit
def kernel(x, gate_w, gate_b):
    weights, indices = pl.pallas_call(
        _router_body,
        in_specs=[
            pl.BlockSpec(memory_space=pl.ANY),
            pl.BlockSpec((E, D), lambda: (0, 0)),
            pl.BlockSpec((1, E), lambda: (0, 0)),
        ],
        out_specs=[
            pl.BlockSpec((B, K), lambda: (0, 0)),
            pl.BlockSpec((B, K), lambda: (0, 0)),
        ],
        out_shape=[
            jax.ShapeDtypeStruct((B, K), jnp.float32),
            jax.ShapeDtypeStruct((B, K), jnp.int32),
        ],
        scratch_shapes=[
            pltpu.VMEM((NBUF, ROWS, D), jnp.float32),
            pltpu.VMEM((B, D), jnp.float32),
            pltpu.SemaphoreType.DMA((NBUF,)),
        ],
    )(x.reshape(B * S, D), gate_w, gate_b.reshape(1, E))
    return weights, indices
